# Initial kernel scaffold; baseline (speedup 1.0000x reference)
import functools

import jax
import jax.numpy as jnp
from jax import lax
from jax.experimental import pallas as pl
from jax.experimental.pallas import tpu as pltpu
from jax.experimental.pallas import tpu_sc as plsc

VOCAB = 100000
HIDDEN = 64
RANK = 16
BATCH = 4096
SEQ = 200
N = BATCH * SEQ

NC = 2
NS = 16
NW = NC * NS
ROWS_PER_W = N // NW
CH = 128
NCHUNK = ROWS_PER_W // CH


VBLK = 1000


def _fuse_body(a_ref, b_ref, base_ref, out_ref):
    out_ref[...] = base_ref[...] + jnp.dot(
        a_ref[...], b_ref[...], preferred_element_type=jnp.float32
    )


def _fused_table(base_table, lora_A, lora_B):
    return pl.pallas_call(
        _fuse_body,
        grid=(VOCAB // VBLK,),
        in_specs=[
            pl.BlockSpec((VBLK, RANK), lambda i: (i, 0)),
            pl.BlockSpec((RANK, HIDDEN), lambda i: (0, 0)),
            pl.BlockSpec((VBLK, HIDDEN), lambda i: (i, 0)),
        ],
        out_specs=pl.BlockSpec((VBLK, HIDDEN), lambda i: (i, 0)),
        out_shape=jax.ShapeDtypeStruct((VOCAB, HIDDEN), jnp.float32),
    )(lora_A, lora_B, base_table)


_mesh = plsc.VectorSubcoreMesh(
    core_axis_name="c", subcore_axis_name="s", num_cores=NC, num_subcores=NS
)


@functools.partial(
    pl.kernel,
    out_type=jax.ShapeDtypeStruct((N, HIDDEN), jnp.float32),
    mesh=_mesh,
    scratch_types=[
        pltpu.VMEM((NCHUNK, CH), jnp.int32),
        pltpu.VMEM((CH, HIDDEN), jnp.float32),
        pltpu.SemaphoreType.DMA,
    ],
)
def _sc_gather(table_hbm, idx_hbm, out_hbm, idx_v, row_v, sem):
    wid = lax.axis_index("s") * NC + lax.axis_index("c")
    base = wid * ROWS_PER_W
    pltpu.sync_copy(idx_hbm.at[wid], idx_v)

    def chunk(j, carry):
        pltpu.async_copy(table_hbm.at[idx_v.at[j]], row_v, sem).wait()
        pltpu.sync_copy(row_v, out_hbm.at[pl.ds(base + j * CH, CH)])
        return carry

    lax.fori_loop(0, NCHUNK, chunk, 0)


def kernel(input_ids, base_table, lora_A, lora_B):
    fused = _fused_table(base_table, lora_A, lora_B)
    ids = input_ids.reshape(-1).astype(jnp.int32).reshape(NW, NCHUNK, CH)
    out = _sc_gather(fused, ids)
    return out.reshape(input_ids.shape[0], input_ids.shape[1], HIDDEN)


# speedup vs baseline: 5.7987x; 5.7987x over previous
import functools

import jax
import jax.numpy as jnp
from jax import lax
from jax.experimental import pallas as pl
from jax.experimental.pallas import tpu as pltpu
from jax.experimental.pallas import tpu_sc as plsc

VOCAB = 100000
HIDDEN = 64
RANK = 16
BATCH = 4096
SEQ = 200
N = BATCH * SEQ

NC = 2
NS = 16
NW = NC * NS
ROWS_PER_W = N // NW
CH = 128
NCHUNK = ROWS_PER_W // CH


VBLK = 1000


def _fuse_body(a_ref, b_ref, base_ref, out_ref):
    out_ref[...] = base_ref[...] + jnp.dot(
        a_ref[...], b_ref[...], preferred_element_type=jnp.float32
    )


def _fused_table(base_table, lora_A, lora_B):
    return pl.pallas_call(
        _fuse_body,
        grid=(VOCAB // VBLK,),
        in_specs=[
            pl.BlockSpec((VBLK, RANK), lambda i: (i, 0)),
            pl.BlockSpec((RANK, HIDDEN), lambda i: (0, 0)),
            pl.BlockSpec((VBLK, HIDDEN), lambda i: (i, 0)),
        ],
        out_specs=pl.BlockSpec((VBLK, HIDDEN), lambda i: (i, 0)),
        out_shape=jax.ShapeDtypeStruct((VOCAB, HIDDEN), jnp.float32),
    )(lora_A, lora_B, base_table)


_mesh = plsc.VectorSubcoreMesh(
    core_axis_name="c", subcore_axis_name="s", num_cores=NC, num_subcores=NS
)


@functools.partial(
    pl.kernel,
    out_type=jax.ShapeDtypeStruct((N, HIDDEN), jnp.float32),
    mesh=_mesh,
    scratch_types=[
        pltpu.VMEM((NCHUNK, CH), jnp.int32),
        pltpu.VMEM((CH, HIDDEN), jnp.float32),
        pltpu.SemaphoreType.DMA,
    ],
    compiler_params=pltpu.CompilerParams(use_tc_tiling_on_sc=False),
)
def _sc_gather(table_hbm, idx_hbm, out_hbm, idx_v, row_v, sem):
    wid = lax.axis_index("s") * NC + lax.axis_index("c")
    base = wid * ROWS_PER_W
    pltpu.sync_copy(idx_hbm.at[wid], idx_v)

    def chunk(j, carry):
        pltpu.async_copy(table_hbm.at[idx_v.at[j]], row_v, sem).wait()
        pltpu.sync_copy(row_v, out_hbm.at[pl.ds(base + j * CH, CH)])
        return carry

    lax.fori_loop(0, NCHUNK, chunk, 0)


def kernel(input_ids, base_table, lora_A, lora_B):
    fused = _fused_table(base_table, lora_A, lora_B)
    ids = input_ids.reshape(-1).astype(jnp.int32).reshape(NW, NCHUNK, CH)
    out = _sc_gather(fused, ids)
    return out.reshape(input_ids.shape[0], input_ids.shape[1], HIDDEN)


# speedup vs baseline: 6.7491x; 1.1639x over previous
import functools

import jax
import jax.numpy as jnp
from jax import lax
from jax.experimental import pallas as pl
from jax.experimental.pallas import tpu as pltpu
from jax.experimental.pallas import tpu_sc as plsc

VOCAB = 100000
HIDDEN = 64
RANK = 16
BATCH = 4096
SEQ = 200
N = BATCH * SEQ

NC = 2
NS = 16
NW = NC * NS
ROWS_PER_W = N // NW
CH = 128
NCHUNK = ROWS_PER_W // CH
NBUF = 4
NGROUP = NCHUNK // NBUF


VBLK = 1000


def _fuse_body(a_ref, b_ref, base_ref, out_ref):
    out_ref[...] = base_ref[...] + jnp.dot(
        a_ref[...], b_ref[...], preferred_element_type=jnp.float32
    )


def _fused_table(base_table, lora_A, lora_B):
    return pl.pallas_call(
        _fuse_body,
        grid=(VOCAB // VBLK,),
        in_specs=[
            pl.BlockSpec((VBLK, RANK), lambda i: (i, 0)),
            pl.BlockSpec((RANK, HIDDEN), lambda i: (0, 0)),
            pl.BlockSpec((VBLK, HIDDEN), lambda i: (i, 0)),
        ],
        out_specs=pl.BlockSpec((VBLK, HIDDEN), lambda i: (i, 0)),
        out_shape=jax.ShapeDtypeStruct((VOCAB, HIDDEN), jnp.float32),
    )(lora_A, lora_B, base_table)


_mesh = plsc.VectorSubcoreMesh(
    core_axis_name="c", subcore_axis_name="s", num_cores=NC, num_subcores=NS
)


@functools.partial(
    pl.kernel,
    out_type=jax.ShapeDtypeStruct((N, HIDDEN), jnp.float32),
    mesh=_mesh,
    scratch_types=[
        pltpu.VMEM((NCHUNK, CH), jnp.int32),
        [pltpu.VMEM((CH, HIDDEN), jnp.float32) for _ in range(NBUF)],
        [pltpu.SemaphoreType.DMA for _ in range(NBUF)],
        [pltpu.SemaphoreType.DMA for _ in range(NBUF)],
    ],
    compiler_params=pltpu.CompilerParams(use_tc_tiling_on_sc=False),
)
def _sc_gather(table_hbm, idx_hbm, out_hbm, idx_v, bufs, gsems, osems):
    wid = lax.axis_index("s") * NC + lax.axis_index("c")
    base = wid * ROWS_PER_W
    pltpu.sync_copy(idx_hbm.at[wid], idx_v)

    def start_gather(j, b):
        pltpu.async_copy(table_hbm.at[idx_v.at[j]], bufs[b], gsems[b])

    def wait_gather(b):
        pltpu.make_async_copy(table_hbm.at[idx_v.at[0]], bufs[b], gsems[b]).wait()

    def start_out(j, b):
        pltpu.async_copy(bufs[b], out_hbm.at[pl.ds(base + j * CH, CH)], osems[b])

    def wait_out(b):
        pltpu.make_async_copy(
            bufs[b], out_hbm.at[pl.ds(base, CH)], osems[b]
        ).wait()

    for b in range(NBUF):
        start_gather(b, b)

    def group(g, carry):
        for b in range(NBUF):
            wait_gather(b)
            start_out(g * NBUF + b, b)
        for b in range(NBUF):
            nxt = (g + 1) * NBUF + b

            @pl.when(nxt < NCHUNK)
            def _():
                wait_out(b)
                start_gather(nxt, b)

        return carry

    lax.fori_loop(0, NGROUP, group, 0)
    for b in range(NBUF):
        wait_out(b)


def kernel(input_ids, base_table, lora_A, lora_B):
    fused = _fused_table(base_table, lora_A, lora_B)
    ids = input_ids.reshape(-1).astype(jnp.int32).reshape(NW, NCHUNK, CH)
    out = _sc_gather(fused, ids)
    return out.reshape(input_ids.shape[0], input_ids.shape[1], HIDDEN)


# speedup vs baseline: 7.0675x; 1.0472x over previous
import functools

import jax
import jax.numpy as jnp
from jax import lax
from jax.experimental import pallas as pl
from jax.experimental.pallas import tpu as pltpu
from jax.experimental.pallas import tpu_sc as plsc

VOCAB = 100000
HIDDEN = 64
RANK = 16
BATCH = 4096
SEQ = 200
N = BATCH * SEQ

VBLK = 1024
NBLK = -(-VOCAB // VBLK)
TROWS = NBLK * (VBLK // 2)

NC = 2
NS = 16
NW = NC * NS
CH = 128
NCHUNK = N // NW // CH
NBUF = 4
NGROUP = NCHUNK // NBUF


def _fuse_body(a_ref, b_ref, base_ref, out_ref):
    res = base_ref[...] + jnp.dot(
        a_ref[...], b_ref[...], preferred_element_type=jnp.float32
    )
    out_ref[:, 0:64] = res[0 : VBLK // 2]
    out_ref[:, 64:128] = res[VBLK // 2 : VBLK]


def _fused_table(base_table, lora_A, lora_B):
    return pl.pallas_call(
        _fuse_body,
        grid=(NBLK,),
        in_specs=[
            pl.BlockSpec((VBLK, RANK), lambda i: (i, 0)),
            pl.BlockSpec((RANK, HIDDEN), lambda i: (0, 0)),
            pl.BlockSpec((VBLK, HIDDEN), lambda i: (i, 0)),
        ],
        out_specs=pl.BlockSpec((VBLK // 2, 128), lambda i: (i, 0)),
        out_shape=jax.ShapeDtypeStruct((TROWS, 128), jnp.float32),
    )(lora_A, lora_B, base_table)


def _remap_body(ids_ref, out_ref):
    v = ids_ref[...]
    out_ref[...] = (v & (-1024)) | ((v & 511) << 1) | ((v >> 9) & 1)


def _remap_ids(ids32):
    blk = 512
    return pl.pallas_call(
        _remap_body,
        grid=(BATCH // blk,),
        in_specs=[pl.BlockSpec((blk, SEQ), lambda i: (i, 0))],
        out_specs=pl.BlockSpec((blk, SEQ), lambda i: (i, 0)),
        out_shape=jax.ShapeDtypeStruct((BATCH, SEQ), jnp.int32),
    )(ids32)


_mesh = plsc.VectorSubcoreMesh(
    core_axis_name="c", subcore_axis_name="s", num_cores=NC, num_subcores=NS
)


@functools.partial(
    pl.kernel,
    out_type=jax.ShapeDtypeStruct((N, HIDDEN), jnp.float32),
    mesh=_mesh,
    scratch_types=[
        pltpu.VMEM((NCHUNK, CH), jnp.int32),
        [pltpu.VMEM((CH, HIDDEN), jnp.float32) for _ in range(NBUF)],
        [pltpu.SemaphoreType.DMA for _ in range(NBUF)],
        [pltpu.SemaphoreType.DMA for _ in range(NBUF)],
    ],
    compiler_params=pltpu.CompilerParams(use_tc_tiling_on_sc=False),
)
def _sc_gather(table_hbm, idx_hbm, out_hbm, idx_v, bufs, gsems, osems):
    wid = lax.axis_index("s") * NC + lax.axis_index("c")
    base = wid * NCHUNK * CH
    pltpu.sync_copy(idx_hbm.at[pl.ds(wid * NCHUNK, NCHUNK)], idx_v)

    def start_gather(j, b):
        pltpu.async_copy(table_hbm.at[idx_v.at[j]], bufs[b], gsems[b])

    def wait_gather(b):
        pltpu.make_async_copy(table_hbm.at[idx_v.at[0]], bufs[b], gsems[b]).wait()

    def start_out(j, b):
        pltpu.async_copy(bufs[b], out_hbm.at[pl.ds(base + j * CH, CH)], osems[b])

    def wait_out(b):
        pltpu.make_async_copy(bufs[b], out_hbm.at[pl.ds(base, CH)], osems[b]).wait()

    for b in range(NBUF):
        start_gather(b, b)

    def group(g, carry):
        for b in range(NBUF):
            wait_gather(b)
            start_out(g * NBUF + b, b)
        for b in range(NBUF):
            nxt = (g + 1) * NBUF + b

            @pl.when(nxt < NCHUNK)
            def _():
                wait_out(b)
                start_gather(nxt, b)

        return carry

    lax.fori_loop(0, NGROUP, group, 0)
    for b in range(NBUF):
        wait_out(b)


def kernel(input_ids, base_table, lora_A, lora_B):
    tab = _fused_table(base_table, lora_A, lora_B)
    u = _remap_ids(input_ids.astype(jnp.int32)).reshape(N // CH, CH)
    out = _sc_gather(tab.reshape(2 * TROWS, HIDDEN), u)
    return out.reshape(input_ids.shape[0], input_ids.shape[1], HIDDEN)


# speedup vs baseline: 7.8287x; 1.1077x over previous
import functools

import jax
import jax.numpy as jnp
from jax import lax
from jax.experimental import pallas as pl
from jax.experimental.pallas import tpu as pltpu
from jax.experimental.pallas import tpu_sc as plsc

VOCAB = 100000
HIDDEN = 64
RANK = 16
BATCH = 4096
SEQ = 200
N = BATCH * SEQ

VBLK = 1024
NBLK = -(-VOCAB // VBLK)
TROWS = NBLK * (VBLK // 2)

NC = 2
NS = 16
NW = NC * NS
RPW = BATCH // NW
CHA = 128
CHB = SEQ - CHA
NBUF = 4
NGROUP = RPW // NBUF


def _fuse_body(at_ref, b_ref, baset_ref, out_ref):
    delta_t = lax.dot_general(
        b_ref[...], at_ref[...], (((0,), (0,)), ((), ())),
        preferred_element_type=jnp.float32,
    )
    res = jnp.transpose(baset_ref[...] + delta_t)
    out_ref[:, 0:64] = res[0 : VBLK // 2]
    out_ref[:, 64:128] = res[VBLK // 2 : VBLK]


def _fused_table(base_t, lora_a_t, lora_B):
    return pl.pallas_call(
        _fuse_body,
        grid=(NBLK,),
        in_specs=[
            pl.BlockSpec((RANK, VBLK), lambda i: (0, i)),
            pl.BlockSpec((RANK, HIDDEN), lambda i: (0, 0)),
            pl.BlockSpec((HIDDEN, VBLK), lambda i: (0, i)),
        ],
        out_specs=pl.BlockSpec((VBLK // 2, 128), lambda i: (i, 0)),
        out_shape=jax.ShapeDtypeStruct((TROWS, 128), jnp.float32),
    )(lora_a_t, lora_B, base_t)


_RBLK = 512


def _remap_body(idst_ref, outa_ref, outb_ref):
    v = idst_ref[...]
    u = (v & (-1024)) | ((v & 511) << 1) | ((v >> 9) & 1)
    ut = jnp.transpose(u)
    outa_ref[...] = ut[:, 0:CHA]
    outb_ref[...] = jnp.concatenate(
        [ut[:, CHA:SEQ], jnp.zeros((_RBLK, CHA - CHB), jnp.int32)], axis=1
    )


def _remap_ids(ids_t):
    return pl.pallas_call(
        _remap_body,
        grid=(BATCH // _RBLK,),
        in_specs=[pl.BlockSpec((SEQ, _RBLK), lambda i: (0, i))],
        out_specs=[
            pl.BlockSpec((_RBLK, CHA), lambda i: (i, 0)),
            pl.BlockSpec((_RBLK, CHA), lambda i: (i, 0)),
        ],
        out_shape=[
            jax.ShapeDtypeStruct((BATCH, CHA), jnp.int32),
            jax.ShapeDtypeStruct((BATCH, CHA), jnp.int32),
        ],
    )(ids_t)


_mesh = plsc.VectorSubcoreMesh(
    core_axis_name="c", subcore_axis_name="s", num_cores=NC, num_subcores=NS
)


@functools.partial(
    pl.kernel,
    out_type=jax.ShapeDtypeStruct((N, HIDDEN), jnp.float32),
    mesh=_mesh,
    scratch_types=[
        pltpu.VMEM((RPW, CHA), jnp.int32),
        pltpu.VMEM((RPW, CHA), jnp.int32),
        [pltpu.VMEM((CHA, HIDDEN), jnp.float32) for _ in range(NBUF)],
        [pltpu.VMEM((CHB, HIDDEN), jnp.float32) for _ in range(NBUF)],
        [pltpu.SemaphoreType.DMA for _ in range(NBUF)],
        [pltpu.SemaphoreType.DMA for _ in range(NBUF)],
        [pltpu.SemaphoreType.DMA for _ in range(NBUF)],
        [pltpu.SemaphoreType.DMA for _ in range(NBUF)],
    ],
    compiler_params=pltpu.CompilerParams(use_tc_tiling_on_sc=False),
)
def _sc_gather(table_hbm, idxa_hbm, idxb_hbm, out_hbm,
               idxa_v, idxb_v, bufA, bufB, gsA, gsB, osA, osB):
    wid = lax.axis_index("s") * NC + lax.axis_index("c")
    row0 = wid * RPW
    pltpu.sync_copy(idxa_hbm.at[pl.ds(row0, RPW)], idxa_v)
    pltpu.sync_copy(idxb_hbm.at[pl.ds(row0, RPW)], idxb_v)

    def start_gather(r, b):
        pltpu.async_copy(table_hbm.at[idxa_v.at[r]], bufA[b], gsA[b])
        pltpu.async_copy(table_hbm.at[idxb_v.at[r, pl.ds(0, CHB)]], bufB[b], gsB[b])

    def wait_gather(b):
        pltpu.make_async_copy(table_hbm.at[idxa_v.at[0]], bufA[b], gsA[b]).wait()
        pltpu.make_async_copy(
            table_hbm.at[idxb_v.at[0, pl.ds(0, CHB)]], bufB[b], gsB[b]
        ).wait()

    def start_out(r, b):
        t0 = (row0 + r) * SEQ
        pltpu.async_copy(bufA[b], out_hbm.at[pl.ds(t0, CHA)], osA[b])
        pltpu.async_copy(bufB[b], out_hbm.at[pl.ds(t0 + CHA, CHB)], osB[b])

    def wait_out(b):
        pltpu.make_async_copy(bufA[b], out_hbm.at[pl.ds(0, CHA)], osA[b]).wait()
        pltpu.make_async_copy(bufB[b], out_hbm.at[pl.ds(0, CHB)], osB[b]).wait()

    for b in range(NBUF):
        start_gather(b, b)

    def group(g, carry):
        for b in range(NBUF):
            wait_gather(b)
            start_out(g * NBUF + b, b)
        for b in range(NBUF):
            nr = (g + 1) * NBUF + b

            @pl.when(nr < RPW)
            def _():
                wait_out(b)
                start_gather(nr, b)

        return carry

    lax.fori_loop(0, NGROUP, group, 0)
    for b in range(NBUF):
        wait_out(b)


def kernel(input_ids, base_table, lora_A, lora_B):
    tab = _fused_table(base_table.T, lora_A.T, lora_B)
    ua, ub = _remap_ids(input_ids.astype(jnp.int32).T)
    out = _sc_gather(tab.reshape(2 * TROWS, HIDDEN), ua, ub)
    return out.reshape(input_ids.shape[0], input_ids.shape[1], HIDDEN)
